# baseline (speedup 1.0000x reference)
import functools

import jax
import jax.numpy as jnp
from jax import lax
from jax.experimental import pallas as pl
from jax.experimental.pallas import tpu as pltpu
from jax.experimental.pallas import tpu_sc as plsc

GLOVE_DIM = 300
D_MODEL = 768
SLICE_W = 128
N_SLICES = 3
DIM_PAD = SLICE_W * N_SLICES


def _make_sc_gather(num_rows: int):
    info = plsc.get_sparse_core_info()
    nc, ns = info.num_cores, info.num_subcores
    nw = nc * ns
    assert num_rows % (8 * nw) == 0
    b_per_w = num_rows // nw
    chunk = 80
    assert b_per_w % chunk == 0 and chunk % 8 == 0
    n_chunks = b_per_w // chunk

    mesh = plsc.VectorSubcoreMesh(core_axis_name="c", subcore_axis_name="s")

    @functools.partial(
        pl.kernel,
        mesh=mesh,
        out_type=jax.ShapeDtypeStruct((num_rows, SLICE_W), jnp.float32),
        scratch_types=[
            pltpu.VMEM((2, chunk), jnp.int32),
            pltpu.VMEM((2, chunk, SLICE_W), jnp.float32),
            pltpu.SemaphoreType.DMA,
            pltpu.SemaphoreType.DMA,
        ],
    )
    def gather(tab_hbm, idx_hbm, out_hbm, idx_v, rows_v, sem0, sem1):
        wid = lax.axis_index("s") * nc + lax.axis_index("c")
        base = wid * b_per_w
        sems = (sem0, sem1)

        def fire(g, buf):
            off = base + g * chunk
            pltpu.sync_copy(idx_hbm.at[pl.ds(off, chunk)], idx_v.at[buf])
            pltpu.async_copy(
                tab_hbm.at[idx_v.at[buf]], rows_v.at[buf], sems[buf]
            )

        def drain_write(g, buf):
            pltpu.make_async_copy(
                tab_hbm.at[idx_v.at[buf]], rows_v.at[buf], sems[buf]
            ).wait()
            pltpu.sync_copy(rows_v.at[buf], out_hbm.at[pl.ds(base + g * chunk, chunk)])

        fire(0, 0)

        def body(t, _):
            g = 2 * t

            @pl.when(g + 1 < n_chunks)
            def _():
                fire(g + 1, 1)

            drain_write(g, 0)

            @pl.when(g + 1 < n_chunks)
            def _():
                @pl.when(g + 2 < n_chunks)
                def _():
                    fire(g + 2, 0)

                drain_write(g + 1, 1)

            return 0

        lax.fori_loop(0, (n_chunks + 1) // 2, body, 0)

    return gather


def _mm_body(a0_ref, a1_ref, a2_ref, w_ref, b_ref, o_ref):
    w = w_ref[...].astype(jnp.bfloat16)
    acc = jnp.dot(
        a0_ref[...].astype(jnp.bfloat16),
        w[:SLICE_W],
        preferred_element_type=jnp.float32,
    )
    acc += jnp.dot(
        a1_ref[...].astype(jnp.bfloat16),
        w[SLICE_W : 2 * SLICE_W],
        preferred_element_type=jnp.float32,
    )
    acc += jnp.dot(
        a2_ref[...].astype(jnp.bfloat16),
        w[2 * SLICE_W :],
        preferred_element_type=jnp.float32,
    )
    o_ref[...] = (acc + b_ref[...]).reshape(1, -1, D_MODEL)


def _matmul_tc(embs, wp, b, batch, hist):
    a_spec = pl.BlockSpec((batch, SLICE_W), lambda i: (i, 0))
    return pl.pallas_call(
        _mm_body,
        grid=(hist,),
        in_specs=[
            a_spec,
            a_spec,
            a_spec,
            pl.BlockSpec((DIM_PAD, D_MODEL), lambda i: (0, 0)),
            pl.BlockSpec((1, D_MODEL), lambda i: (0, 0)),
        ],
        out_specs=pl.BlockSpec((1, batch, D_MODEL), lambda i: (i, 0, 0)),
        out_shape=jax.ShapeDtypeStruct((hist, batch, D_MODEL), jnp.float32),
    )(*embs, wp, b.reshape(1, D_MODEL))


def kernel(x, glove_table, W, b):
    batch, hist = x.shape
    idx = x.T.astype(jnp.int32).reshape(-1)
    gather = _make_sc_gather(idx.shape[0])
    embs = []
    for t in range(N_SLICES):
        lo = t * SLICE_W
        hi = min(GLOVE_DIM, lo + SLICE_W)
        tab = glove_table[:, lo:hi]
        if hi - lo < SLICE_W:
            tab = jnp.pad(tab, ((0, 0), (0, SLICE_W - (hi - lo))))
        embs.append(gather(tab, idx))
    wp = jnp.pad(W, ((0, DIM_PAD - GLOVE_DIM), (0, 0)))
    out_t = _matmul_tc(embs, wp, b, batch, hist)
    return jnp.transpose(out_t, (1, 0, 2))
